# baseline (speedup 1.0000x reference)
import functools

import jax
import jax.numpy as jnp
from jax import lax
from jax.experimental import pallas as pl
from jax.experimental.pallas import tpu as pltpu
from jax.experimental.pallas import tpu_sc as plsc

EMBED = 64
NC = 2
NS = 16
NW = NC * NS
CH = 128
JJ = 4
GR = CH * JJ


def _emb_body(tot, idx_hbm, table_hbm, out_hbm, idx_v, rows_v, gsem):
    pw = tot // NW
    ng = pw // GR
    wid = lax.axis_index("s") * NC + lax.axis_index("c")
    row0 = wid * pw

    def fire(g, b):
        pltpu.sync_copy(idx_hbm.at[pl.ds(row0 + g * GR, GR)], idx_v.at[b])
        for j in range(JJ):
            pltpu.async_copy(
                table_hbm.at[idx_v.at[b, pl.ds(j * CH, CH)]],
                rows_v.at[b, pl.ds(j * CH, CH)],
                gsem,
            )

    def drain(b):
        pltpu.make_async_copy(
            table_hbm.at[pl.ds(0, GR)], rows_v.at[b], gsem
        ).wait()

    def store(g, b):
        pltpu.sync_copy(rows_v.at[b], out_hbm.at[pl.ds(row0 + g * GR, GR)])

    fire(0, 0)

    def body(p, carry):
        g0 = p * 2
        fire(g0 + 1, 1)
        drain(0)
        store(g0, 0)
        fire(g0 + 2, 0)
        drain(1)
        store(g0 + 1, 1)
        return carry

    lax.fori_loop(0, ng // 2 - 1, body, 0, unroll=False)

    g0 = ng - 2
    fire(g0 + 1, 1)
    drain(0)
    store(g0, 0)
    drain(1)
    store(g0 + 1, 1)


def kernel(indices, table):
    b, s = indices.shape
    tot = b * s
    idx_flat = indices.astype(jnp.int32).reshape(tot)
    grid_kernel = pl.kernel(
        functools.partial(_emb_body, tot),
        out_type=jax.ShapeDtypeStruct((tot, EMBED), jnp.float32),
        mesh=plsc.VectorSubcoreMesh(core_axis_name="c", subcore_axis_name="s"),
        scratch_types=[
            pltpu.VMEM((2, GR), jnp.int32),
            pltpu.VMEM((2, GR, EMBED), jnp.float32),
            pltpu.SemaphoreType.DMA,
        ],
        compiler_params=pltpu.CompilerParams(use_tc_tiling_on_sc=False),
    )
    out = grid_kernel(idx_flat, table)
    return out.reshape(b, s, EMBED)
